# Initial kernel scaffold; baseline (speedup 1.0000x reference)
import jax
import jax.numpy as jnp
from jax.experimental import pallas as pl


def kernel(x, gate_w, wi_gate, wi_up, wo):
    raise NotImplementedError("write your pallas kernel here")


# speedup vs baseline: 2.3226x; 2.3226x over previous
import functools

import jax
import jax.numpy as jnp
from jax import lax
from jax.experimental import pallas as pl
from jax.experimental.pallas import tpu as pltpu
from jax.experimental.pallas import tpu_sc as plsc

N = 8192
D = 768
F = 3072
E = 8
CAP = 1280
NROWS = E * CAP
SENT = 1 << 20
TB = 1024
NB = N // TB
FB = 512
NFB = F // FB
EPAD = 128

NW = 32
RPT = NROWS // NW
TPT = N // NW
PAIR_CHUNK = 2048
ROW_CHUNK = 40
TOK_CHUNK = 32


def _router_body(x_ref, gw_ref, s0_ref, s1_ref, w0_ref, w1_ref, carry_ref):
    pid = pl.program_id(0)

    @pl.when(pid == 0)
    def _():
        carry_ref[...] = jnp.zeros((EPAD, 128), jnp.float32)

    xb = x_ref[...]
    gw = gw_ref[...]
    logits = lax.dot_general(gw, xb, (((1,), (1,)), ((), ())),
                             preferred_element_type=jnp.float32)
    sub = lax.broadcasted_iota(jnp.int32, (EPAD, TB), 0)
    neg = jnp.float32(-1e30)
    logits = jnp.where(sub < E, logits, neg)
    m0 = jnp.max(logits, axis=0, keepdims=True)
    i0 = jnp.argmax(logits, axis=0, keepdims=True)
    masked2 = jnp.where(sub == i0, neg, logits)
    m1 = jnp.max(masked2, axis=0, keepdims=True)
    i1 = jnp.argmax(masked2, axis=0, keepdims=True)

    oh0 = (sub == i0).astype(jnp.float32)
    oh1 = (sub == i1).astype(jnp.float32)
    cnt = oh0 + oh1
    rr = lax.broadcasted_iota(jnp.int32, (TB, TB), 0)
    cc = lax.broadcasted_iota(jnp.int32, (TB, TB), 1)
    tri = (rr < cc).astype(jnp.float32)
    pre = lax.dot_general(cnt, tri, (((1,), (0,)), ((), ())),
                          preferred_element_type=jnp.float32)
    car = carry_ref[...][:, :1]
    pos0f = jnp.sum((pre + car) * oh0, axis=0, keepdims=True)
    pos1f = jnp.sum((pre + car) * oh1, axis=0, keepdims=True)
    carry_ref[...] = carry_ref[...] + jnp.sum(cnt, axis=1, keepdims=True)

    pos0 = pos0f.astype(jnp.int32)
    pos1 = pos1f.astype(jnp.int32)
    v0 = pos0 < CAP
    v1 = pos1 < CAP
    slot0 = jnp.where(v0, i0 * CAP + pos0, SENT)
    slot1 = jnp.where(v1, i1 * CAP + pos1, SENT)
    w0 = jnp.where(v0, jax.nn.sigmoid(m0 - m1), 0.0)
    w1 = jnp.where(v1, jax.nn.sigmoid(m1 - m0), 0.0)

    s0_ref[...] = slot0.reshape(1, 1, TB)
    s1_ref[...] = slot1.reshape(1, 1, TB)
    w0_ref[...] = w0.reshape(1, 1, TB)
    w1_ref[...] = w1.reshape(1, 1, TB)


def _router(x, gw_pad):
    iidx = jnp.int32
    out_shapes = [
        jax.ShapeDtypeStruct((NB, 1, TB), iidx),
        jax.ShapeDtypeStruct((NB, 1, TB), iidx),
        jax.ShapeDtypeStruct((NB, 1, TB), jnp.float32),
        jax.ShapeDtypeStruct((NB, 1, TB), jnp.float32),
    ]
    out_spec = pl.BlockSpec((1, 1, TB), lambda i: (i, 0, 0))
    return pl.pallas_call(
        _router_body,
        grid=(NB,),
        in_specs=[
            pl.BlockSpec((TB, D), lambda i: (i, 0)),
            pl.BlockSpec((EPAD, D), lambda i: (0, 0)),
        ],
        out_specs=[out_spec] * 4,
        out_shape=out_shapes,
        scratch_shapes=[pltpu.VMEM((EPAD, 128), jnp.float32)],
    )(x, gw_pad)


def _dispatch_body(s0_hbm, s1_hbm, x_hbm, disp_hbm,
                   map_v, sbuf_v, rows_v, sem):
    wid = lax.axis_index("s") * 2 + lax.axis_index("c")
    lo = wid * RPT

    for i in range(RPT // 16):
        map_v[pl.ds(i * 16, 16)] = jnp.zeros((16,), jnp.int32)

    lane = lax.iota(jnp.int32, 16)
    for src in (s0_hbm, s1_hbm):
        for ck in range(N // PAIR_CHUNK):
            pltpu.sync_copy(src.at[pl.ds(ck * PAIR_CHUNK, PAIR_CHUNK)], sbuf_v)
            base_tok = ck * PAIR_CHUNK

            def grp(g, _):
                s = sbuf_v[pl.ds(g * 16, 16)]
                tok = base_tok + g * 16 + lane
                m = (s >= lo) & (s < lo + RPT)
                loc = jnp.clip(s - lo, 0, RPT - 1)
                plsc.store_scatter(map_v, [loc], tok, mask=m)
                return 0

            lax.fori_loop(0, PAIR_CHUNK // 16, grp, 0)

    for ch in range(RPT // ROW_CHUNK):
        idx = map_v.at[pl.ds(ch * ROW_CHUNK, ROW_CHUNK)]
        pltpu.async_copy(x_hbm.at[idx], rows_v, sem).wait()
        pltpu.sync_copy(rows_v, disp_hbm.at[pl.ds(lo + ch * ROW_CHUNK, ROW_CHUNK)])


def _dispatch(s0, s1, x):
    mesh = plsc.VectorSubcoreMesh(core_axis_name="c", subcore_axis_name="s")
    fn = functools.partial(
        pl.kernel,
        mesh=mesh,
        out_type=jax.ShapeDtypeStruct((NROWS, D), jnp.float32),
        scratch_types=[
            pltpu.VMEM((RPT,), jnp.int32),
            pltpu.VMEM((PAIR_CHUNK,), jnp.int32),
            pltpu.VMEM((ROW_CHUNK, D), jnp.float32),
            pltpu.SemaphoreType.DMA,
        ],
        compiler_params=pltpu.CompilerParams(needs_layout_passes=False),
    )(_dispatch_body)
    return fn(s0, s1, x)


def _ffn_body(disp_ref, wig_ref, wiu_ref, wo_ref, out_ref):
    fb = pl.program_id(1)
    xb = disp_ref[...]
    g = lax.dot_general(xb, wig_ref[0], (((1,), (1,)), ((), ())),
                        preferred_element_type=jnp.float32)
    u = lax.dot_general(xb, wiu_ref[0], (((1,), (1,)), ((), ())),
                        preferred_element_type=jnp.float32)
    h = g * jax.nn.sigmoid(g) * u
    part = lax.dot_general(h, wo_ref[0], (((1,), (1,)), ((), ())),
                           preferred_element_type=jnp.float32)

    @pl.when(fb == 0)
    def _():
        out_ref[...] = part

    @pl.when(fb > 0)
    def _():
        out_ref[...] = out_ref[...] + part


def _ffn(disp, wig, wiu, wo):
    return pl.pallas_call(
        _ffn_body,
        grid=(E, NFB),
        in_specs=[
            pl.BlockSpec((CAP, D), lambda e, f: (e, 0)),
            pl.BlockSpec((1, FB, D), lambda e, f: (e, f, 0)),
            pl.BlockSpec((1, FB, D), lambda e, f: (e, f, 0)),
            pl.BlockSpec((1, D, FB), lambda e, f: (e, 0, f)),
        ],
        out_specs=pl.BlockSpec((CAP, D), lambda e, f: (e, 0)),
        out_shape=jax.ShapeDtypeStruct((NROWS, D), jnp.float32),
    )(disp, wig, wiu, wo)


def _combine_body(eout_hbm, s0_hbm, s1_hbm, w0_hbm, w1_hbm, y_hbm,
                  s0_v, s1_v, w0_v, w1_v, i0_v, i1_v, r0_v, r1_v, o_v, sem):
    wid = lax.axis_index("s") * 2 + lax.axis_index("c")
    base = wid * TPT

    pltpu.sync_copy(s0_hbm.at[pl.ds(base, TPT)], s0_v)
    pltpu.sync_copy(s1_hbm.at[pl.ds(base, TPT)], s1_v)
    pltpu.sync_copy(w0_hbm.at[pl.ds(base, TPT)], w0_v)
    pltpu.sync_copy(w1_hbm.at[pl.ds(base, TPT)], w1_v)

    cap_idx = jnp.full((16,), NROWS - 1, jnp.int32)
    for i in range(TPT // 16):
        s0_v[pl.ds(i * 16, 16)] = jnp.minimum(s0_v[pl.ds(i * 16, 16)], cap_idx)
        s1_v[pl.ds(i * 16, 16)] = jnp.minimum(s1_v[pl.ds(i * 16, 16)], cap_idx)

    for ch in range(TPT // TOK_CHUNK):
        for g in range(TOK_CHUNK // 16):
            i0_v[pl.ds(g * 16, 16)] = s0_v[pl.ds(ch * TOK_CHUNK + g * 16, 16)]
            i1_v[pl.ds(g * 16, 16)] = s1_v[pl.ds(ch * TOK_CHUNK + g * 16, 16)]
        pltpu.async_copy(eout_hbm.at[i0_v], r0_v, sem).wait()
        pltpu.async_copy(eout_hbm.at[i1_v], r1_v, sem).wait()

        def tok(t, _):
            idxv = jnp.full((16,), ch * TOK_CHUNK, jnp.int32) + t
            w0s = plsc.load_gather(w0_v, [idxv])
            w1s = plsc.load_gather(w1_v, [idxv])
            for dd in range(D // 16):
                sl = pl.ds(dd * 16, 16)
                o_v[t, sl] = r0_v[t, sl] * w0s + r1_v[t, sl] * w1s
            return 0

        lax.fori_loop(0, TOK_CHUNK, tok, 0)
        pltpu.sync_copy(o_v, y_hbm.at[pl.ds(base + ch * TOK_CHUNK, TOK_CHUNK)])


def _combine(eout, s0, s1, w0, w1):
    mesh = plsc.VectorSubcoreMesh(core_axis_name="c", subcore_axis_name="s")
    fn = functools.partial(
        pl.kernel,
        mesh=mesh,
        out_type=jax.ShapeDtypeStruct((N, D), jnp.float32),
        scratch_types=[
            pltpu.VMEM((TPT,), jnp.int32),
            pltpu.VMEM((TPT,), jnp.int32),
            pltpu.VMEM((TPT,), jnp.float32),
            pltpu.VMEM((TPT,), jnp.float32),
            pltpu.VMEM((TOK_CHUNK,), jnp.int32),
            pltpu.VMEM((TOK_CHUNK,), jnp.int32),
            pltpu.VMEM((TOK_CHUNK, D), jnp.float32),
            pltpu.VMEM((TOK_CHUNK, D), jnp.float32),
            pltpu.VMEM((TOK_CHUNK, D), jnp.float32),
            pltpu.SemaphoreType.DMA,
        ],
        compiler_params=pltpu.CompilerParams(needs_layout_passes=False),
    )(_combine_body)
    return fn(eout, s0, s1, w0, w1)


def kernel(x, gate_w, wi_gate, wi_up, wo):
    gw_pad = jnp.pad(gate_w, ((0, EPAD - E), (0, 0)))
    s0, s1, w0, w1 = _router(x, gw_pad)
    s0 = s0.reshape(N)
    s1 = s1.reshape(N)
    w0 = w0.reshape(N)
    w1 = w1.reshape(N)
    disp = _dispatch(s0, s1, x)
    eout = _ffn(disp, wi_gate, wi_up, wo)
    return _combine(eout, s0, s1, w0, w1)
